# Initial kernel scaffold; baseline (speedup 1.0000x reference)
import jax
import jax.numpy as jnp
from jax.experimental import pallas as pl


def kernel(x, edge_index, W1, b1, W2, b2, W4, b4):
    raise NotImplementedError("write your pallas kernel here")


# speedup vs baseline: 2.2082x; 2.2082x over previous
import functools

import jax
import jax.numpy as jnp
from jax import lax
from jax.experimental import pallas as pl
from jax.experimental.pallas import tpu as pltpu
from jax.experimental.pallas import tpu_sc as plsc

N = 10000
D = 128
E = 320000
NC = 2
NS = 16
L = 16
NW = NC * NS
EPW = E // NW
K = 40
NBLK = EPW // K
NP = 10240
NR2 = 1280
NT = NP + NR2
RPT = NT // NS
BN = 2000


def _lane_take(v, idx):
    return lax.gather(
        v, idx[:, None],
        lax.GatherDimensionNumbers(
            offset_dims=(), collapsed_slice_dims=(0,), start_index_map=(0,)),
        (1,), mode=lax.GatherScatterMode.PROMISE_IN_BOUNDS)


def _prep_body(x_ref, w1_ref, g_ref, xw1_ref):
    xw = jnp.dot(x_ref[...], w1_ref[...], preferred_element_type=jnp.float32)
    xw1_ref[...] = xw
    g_ref[...] = jnp.concatenate([xw, x_ref[...]], axis=1)


def _prep(x, W1):
    return pl.pallas_call(
        _prep_body,
        grid=(N // BN,),
        in_specs=[
            pl.BlockSpec((BN, D), lambda i: (i, 0)),
            pl.BlockSpec((D, D), lambda i: (0, 0)),
        ],
        out_specs=[
            pl.BlockSpec((BN, 2 * D), lambda i: (i, 0)),
            pl.BlockSpec((BN, D), lambda i: (i, 0)),
        ],
        out_shape=[
            jax.ShapeDtypeStruct((N, 2 * D), jnp.float32),
            jax.ShapeDtypeStruct((N, D), jnp.float32),
        ],
    )(x, W1)


def _combine_body(x_ref, w4_ref, b4_ref, acc_ref, wd_ref, h_ref):
    x = x_ref[...]
    xl = jnp.dot(x, w4_ref[...], preferred_element_type=jnp.float32) + b4_ref[...]
    s = acc_ref[0] + acc_ref[1]
    a2 = wd_ref[0] + wd_ref[1]
    wsum = a2[:, 0:1]
    deg = a2[:, 1:2]
    h_ref[...] = wsum * x + deg * xl - s


def _combine(x, W4, b4, accs, wd):
    return pl.pallas_call(
        _combine_body,
        grid=(N // BN,),
        in_specs=[
            pl.BlockSpec((BN, D), lambda i: (i, 0)),
            pl.BlockSpec((D, D), lambda i: (0, 0)),
            pl.BlockSpec((1, D), lambda i: (0, 0)),
            pl.BlockSpec((NC, BN, D), lambda i: (0, i, 0)),
            pl.BlockSpec((NC, BN, 2), lambda i: (0, i, 0)),
        ],
        out_specs=pl.BlockSpec((BN, D), lambda i: (i, 0)),
        out_shape=jax.ShapeDtypeStruct((N, D), jnp.float32),
    )(x, W4, b4, accs, wd)


def _sc_body(src_hbm, dst_hbm, mix_hbm, g_hbm, xw1_hbm, par_hbm, zero_hbm,
             out_hbm,
             sidx, didx, midx, gsrc, gdst, mall, pvm, acc, sem, sem2):
    c = lax.axis_index("c")
    s = lax.axis_index("s")
    wid = s * NC + c

    pltpu.sync_copy(par_hbm, pvm)
    pltpu.sync_copy(zero_hbm, acc.at[pl.ds(s * RPT, RPT)])

    zv = jnp.zeros((L,), jnp.float32)

    @pl.loop(0, 2 * K)
    def _z(j):
        for v in range(D // L):
            mall[j, pl.ds(v * L, L)] = zv

    plsc.subcore_barrier()

    ebase = wid * EPW
    io = lax.iota(jnp.int32, L)
    b1vs = [pvm[pl.ds(v * L, L)] for v in range(D // L)]
    w2vs = [pvm[pl.ds(D + v * L, L)] for v in range(D // L)]
    b2v = pvm[pl.ds(2 * D, L)]

    @pl.loop(0, NBLK)
    def _blk(blk):
        base = ebase + blk * K
        pltpu.sync_copy(src_hbm.at[pl.ds(base, K)], sidx)
        pltpu.sync_copy(dst_hbm.at[pl.ds(base, K)], didx)
        pltpu.sync_copy(mix_hbm.at[pl.ds(base * 2, 2 * K)], midx)
        cp1 = pltpu.async_copy(g_hbm.at[sidx], gsrc, sem)
        cp2 = pltpu.async_copy(xw1_hbm.at[didx], gdst, sem2)
        cp1.wait()
        cp2.wait()

        @pl.loop(0, K)
        def _edge(j):
            dot = zv
            for v in range(D // L):
                a = gdst[j, pl.ds(v * L, L)]
                b = gsrc[j, pl.ds(v * L, L)]
                r = jnp.maximum(a - b + b1vs[v], 0.0)
                dot = dot + r * w2vs[v]
            for sh in (8, 4, 2, 1):
                dot = dot + _lane_take(dot, io ^ sh)
            wv = 1.0 / (1.0 + jnp.exp(-(dot + b2v)))
            for v in range(D // L):
                xs = gsrc[j, pl.ds(D + v * L, L)]
                mall[j, pl.ds(v * L, L)] = wv * xs
            jhi = jnp.minimum((j // L) * L, K - L)
            dvec = didx[pl.ds(jhi, L)]
            djv = _lane_take(dvec, jnp.broadcast_to(j - jhi, (L,)))
            om = (djv & 7) * 2
            mall[K + j, pl.ds(0, L)] = jnp.where(
                io == om, wv, jnp.where(io == om + 1, 1.0, 0.0))

        pltpu.sync_copy(mall, acc.at[midx], add=True)

    plsc.subcore_barrier()
    pltpu.sync_copy(acc.at[pl.ds(s * RPT, RPT)],
                    out_hbm.at[c, pl.ds(s * RPT, RPT)])


_sc_edges = functools.partial(
    pl.kernel,
    out_type=jax.ShapeDtypeStruct((NC, NT, D), jnp.float32),
    mesh=plsc.VectorSubcoreMesh(core_axis_name="c", subcore_axis_name="s"),
    scratch_types=[
        pltpu.VMEM((K,), jnp.int32),
        pltpu.VMEM((K,), jnp.int32),
        pltpu.VMEM((2 * K,), jnp.int32),
        pltpu.VMEM((K, 2 * D), jnp.float32),
        pltpu.VMEM((K, D), jnp.float32),
        pltpu.VMEM((2 * K, D), jnp.float32),
        pltpu.VMEM((2 * D + L,), jnp.float32),
        pltpu.VMEM_SHARED((NT, D), jnp.float32),
        pltpu.SemaphoreType.DMA,
        pltpu.SemaphoreType.DMA,
    ],
)(_sc_body)


def kernel(x, edge_index, W1, b1, W2, b2, W4, b4):
    g, xw1 = _prep(x, W1)
    src = edge_index[0]
    dst = edge_index[1]
    mix = jnp.concatenate(
        [dst.reshape(-1, K), NP + lax.shift_right_logical(dst, 3).reshape(-1, K)],
        axis=1).reshape(-1)
    par = jnp.concatenate([b1, W2[:, 0], jnp.broadcast_to(b2, (L,))])
    zeros = jnp.zeros((RPT, D), jnp.float32)
    accs = _sc_edges(src, dst, mix, g, xw1, par, zeros)
    wd = accs[:, NP:NP + N // 8, :16].reshape(NC, N, 2)
    return _combine(x, W4, b4.reshape(1, D), accs, wd)


# speedup vs baseline: 2.6610x; 1.2050x over previous
import functools

import jax
import jax.numpy as jnp
from jax import lax
from jax.experimental import pallas as pl
from jax.experimental.pallas import tpu as pltpu
from jax.experimental.pallas import tpu_sc as plsc

N = 10000
D = 128
E = 320000
NC = 2
NS = 16
L = 16
NW = NC * NS
EPW = E // NW
K = 40
NBLK = EPW // K
NP = 10112
NR2 = 640
NT = NP + NR2
RPT = NT // NS
BN = 2000


def _lane_take(v, idx):
    return lax.gather(
        v, idx[:, None],
        lax.GatherDimensionNumbers(
            offset_dims=(), collapsed_slice_dims=(0,), start_index_map=(0,)),
        (1,), mode=lax.GatherScatterMode.PROMISE_IN_BOUNDS)


def _prep_body(x_ref, w1_ref, g_ref, xw1_ref):
    xw = jnp.dot(x_ref[...], w1_ref[...], preferred_element_type=jnp.float32)
    xw1_ref[...] = xw
    g_ref[...] = jnp.concatenate([xw, x_ref[...]], axis=1)


def _prep(x, W1):
    return pl.pallas_call(
        _prep_body,
        grid=(N // BN,),
        in_specs=[
            pl.BlockSpec((BN, D), lambda i: (i, 0)),
            pl.BlockSpec((D, D), lambda i: (0, 0)),
        ],
        out_specs=[
            pl.BlockSpec((BN, 2 * D), lambda i: (i, 0)),
            pl.BlockSpec((BN, D), lambda i: (i, 0)),
        ],
        out_shape=[
            jax.ShapeDtypeStruct((N, 2 * D), jnp.float32),
            jax.ShapeDtypeStruct((N, D), jnp.float32),
        ],
    )(x, W1)


def _combine_body(x_ref, w4_ref, b4_ref, acc_ref, wd_ref, h_ref):
    x = x_ref[...]
    xl = jnp.dot(x, w4_ref[...], preferred_element_type=jnp.float32) + b4_ref[...]
    s = acc_ref[0] + acc_ref[1]
    a2 = wd_ref[0] + wd_ref[1]
    wsum = a2[:, 0:1]
    deg = a2[:, 1:2]
    h_ref[...] = wsum * x + deg * xl - s


def _combine(x, W4, b4, accs, wd):
    return pl.pallas_call(
        _combine_body,
        grid=(N // BN,),
        in_specs=[
            pl.BlockSpec((BN, D), lambda i: (i, 0)),
            pl.BlockSpec((D, D), lambda i: (0, 0)),
            pl.BlockSpec((1, D), lambda i: (0, 0)),
            pl.BlockSpec((NC, BN, D), lambda i: (0, i, 0)),
            pl.BlockSpec((NC, BN, 2), lambda i: (0, i, 0)),
        ],
        out_specs=pl.BlockSpec((BN, D), lambda i: (i, 0)),
        out_shape=jax.ShapeDtypeStruct((N, D), jnp.float32),
    )(x, W4, b4, accs, wd)


def _sc_body(src_hbm, dst_hbm, mix_hbm, g_hbm, xw1_hbm, par_hbm, zero_hbm,
             out_hbm,
             sidx0, sidx1, didx0, didx1, midx0, midx1,
             gsrc0, gsrc1, gdst0, gdst1, mall, pvm, acc,
             sa0, sa1, sb0, sb1):
    c = lax.axis_index("c")
    s = lax.axis_index("s")
    wid = s * NC + c
    sidx = (sidx0, sidx1)
    didx = (didx0, didx1)
    midx = (midx0, midx1)
    gsrc = (gsrc0, gsrc1)
    gdst = (gdst0, gdst1)
    sa = (sa0, sa1)
    sb = (sb0, sb1)

    pltpu.sync_copy(par_hbm, pvm)
    pltpu.sync_copy(zero_hbm, acc.at[pl.ds(s * RPT, RPT)])

    zv = jnp.zeros((L,), jnp.float32)

    @pl.loop(0, 2 * K)
    def _z(j):
        for v in range(D // L):
            mall[j, pl.ds(v * L, L)] = zv

    plsc.subcore_barrier()

    ebase = wid * EPW
    io = lax.iota(jnp.int32, L)
    io16 = io + L
    b1vs = [pvm[pl.ds(v * L, L)] for v in range(D // L)]
    w2vs = [pvm[pl.ds(D + v * L, L)] for v in range(D // L)]
    b2v = pvm[pl.ds(2 * D, L)]

    def load_and_fire(buf, blk):
        base = ebase + blk * K
        pltpu.sync_copy(src_hbm.at[pl.ds(base, K)], sidx[buf])
        pltpu.sync_copy(dst_hbm.at[pl.ds(base, K)], didx[buf])
        pltpu.sync_copy(mix_hbm.at[pl.ds(base * 2, 2 * K)], midx[buf])
        pltpu.async_copy(g_hbm.at[sidx[buf]], gsrc[buf], sa[buf])
        pltpu.async_copy(xw1_hbm.at[didx[buf]], gdst[buf], sb[buf])

    load_and_fire(0, 0)

    @pl.loop(0, NBLK // 2)
    def _blk2(it):
        for b in (0, 1):
            cur = it * 2 + b
            nxt = cur + 1

            @pl.when(nxt < NBLK)
            def _pf():
                load_and_fire(1 - b, nxt)

            pltpu.make_async_copy(g_hbm.at[sidx[b]], gsrc[b], sa[b]).wait()
            pltpu.make_async_copy(xw1_hbm.at[didx[b]], gdst[b], sb[b]).wait()

            gsr = gsrc[b]
            gds = gdst[b]
            ddx = didx[b]

            @pl.loop(0, K)
            def _edge(j):
                dot = zv
                for v in range(D // L):
                    a = gds[j, pl.ds(v * L, L)]
                    bb = gsr[j, pl.ds(v * L, L)]
                    r = jnp.maximum(a - bb + b1vs[v], 0.0)
                    dot = dot + r * w2vs[v]
                for sh in (8, 4, 2, 1):
                    dot = dot + _lane_take(dot, io ^ sh)
                wv = 1.0 / (1.0 + jnp.exp(-(dot + b2v)))
                for v in range(D // L):
                    xs = gsr[j, pl.ds(D + v * L, L)]
                    mall[j, pl.ds(v * L, L)] = wv * xs
                jhi = jnp.minimum((j // L) * L, K - L)
                dvec = ddx[pl.ds(jhi, L)]
                djv = _lane_take(dvec, jnp.broadcast_to(j - jhi, (L,)))
                om = (djv & 15) * 2
                mall[K + j, pl.ds(0, L)] = jnp.where(
                    io == om, wv, jnp.where(io == om + 1, 1.0, 0.0))
                mall[K + j, pl.ds(L, L)] = jnp.where(
                    io16 == om, wv, jnp.where(io16 == om + 1, 1.0, 0.0))

            pltpu.sync_copy(mall, acc.at[midx[b]], add=True)

    plsc.subcore_barrier()
    pltpu.sync_copy(acc.at[pl.ds(s * RPT, RPT)],
                    out_hbm.at[c, pl.ds(s * RPT, RPT)])


_sc_edges = functools.partial(
    pl.kernel,
    out_type=jax.ShapeDtypeStruct((NC, NT, D), jnp.float32),
    mesh=plsc.VectorSubcoreMesh(core_axis_name="c", subcore_axis_name="s"),
    scratch_types=[
        pltpu.VMEM((K,), jnp.int32),
        pltpu.VMEM((K,), jnp.int32),
        pltpu.VMEM((K,), jnp.int32),
        pltpu.VMEM((K,), jnp.int32),
        pltpu.VMEM((2 * K,), jnp.int32),
        pltpu.VMEM((2 * K,), jnp.int32),
        pltpu.VMEM((K, 2 * D), jnp.float32),
        pltpu.VMEM((K, 2 * D), jnp.float32),
        pltpu.VMEM((K, D), jnp.float32),
        pltpu.VMEM((K, D), jnp.float32),
        pltpu.VMEM((2 * K, D), jnp.float32),
        pltpu.VMEM((2 * D + L,), jnp.float32),
        pltpu.VMEM_SHARED((NT, D), jnp.float32),
        pltpu.SemaphoreType.DMA,
        pltpu.SemaphoreType.DMA,
        pltpu.SemaphoreType.DMA,
        pltpu.SemaphoreType.DMA,
    ],
)(_sc_body)


def kernel(x, edge_index, W1, b1, W2, b2, W4, b4):
    g, xw1 = _prep(x, W1)
    src = edge_index[0]
    dst = edge_index[1]
    mix = jnp.concatenate(
        [dst.reshape(-1, K), NP + lax.shift_right_logical(dst, 4).reshape(-1, K)],
        axis=1).reshape(-1)
    par = jnp.concatenate([b1, W2[:, 0], jnp.broadcast_to(b2, (L,))])
    zeros = jnp.zeros((RPT, D), jnp.float32)
    accs = _sc_edges(src, dst, mix, g, xw1, par, zeros)
    wd = accs[:, NP:NP + N // 16, :32].reshape(NC, N, 2)
    return _combine(x, W4, b4.reshape(1, D), accs, wd)
